# Initial kernel scaffold; baseline (speedup 1.0000x reference)
import jax
import jax.numpy as jnp
from jax.experimental import pallas as pl


def kernel(points, core_first, cores_mid, core_last, nodes, W1, b1, W2, b2, W3, b3):
    raise NotImplementedError("write your pallas kernel here")


# speedup vs baseline: 10.6863x; 10.6863x over previous
import jax
import jax.numpy as jnp
from jax import lax
from jax.experimental import pallas as pl
from jax.experimental.pallas import tpu as pltpu
from jax.experimental.pallas import tpu_sc as plsc

B = 16384
D = 26
M = 64
R = 16
H = 52
DM = D - 2
NC = 2
NS = 16
NW = NC * NS
P = B // NW
NG = P // 16
BT = 2048
Q = 4096


def _tc_body(x_ref, w1_ref, b1_ref, w2_ref, b2_ref, w3_ref, b3_ref, nn_ref):
    x = x_ref[...]
    cdims = (((1,), (1,)), ((), ()))
    h = jnp.maximum(
        lax.dot_general(x, w1_ref[...], cdims,
                        preferred_element_type=jnp.float32) + b1_ref[...], 0.0)
    h = jnp.maximum(
        lax.dot_general(h, w2_ref[...], cdims,
                        preferred_element_type=jnp.float32) + b2_ref[...], 0.0)
    nn_ref[...] = jnp.sum(h * w3_ref[...], axis=1, keepdims=True) + b3_ref[0, 0]


def _sc_body(pts_hbm, nn_hbm, lut_hbm, mid_hbm, cf_hbm, cm_hbm, cl_hbm, out_hbm,
             pts_v, nn_v, lut_v, mid_v, cf_v, cl_v, cm_v, v_ref, out_v,
             sem0, sem1):
    cid = lax.axis_index("c")
    sid = lax.axis_index("s")
    wid = sid * NC + cid
    pltpu.sync_copy(pts_hbm.at[wid], pts_v)
    pltpu.sync_copy(nn_hbm.at[wid], nn_v)
    pltpu.sync_copy(lut_hbm, lut_v)
    pltpu.sync_copy(mid_hbm, mid_v)
    pltpu.sync_copy(cf_hbm, cf_v)
    pltpu.sync_copy(cl_hbm, cl_v)
    pltpu.async_copy(cm_hbm.at[0], cm_v.at[pl.ds(0, R * M * R)], sem0)
    iota_d = lax.iota(jnp.int32, 16) * D

    def nearest(off, d):
        x = plsc.load_gather(pts_v, [iota_d + (off * D + d)])
        q = jnp.minimum((x * float(Q)).astype(jnp.int32), Q - 1)
        lo = plsc.load_gather(lut_v, [q])
        mv = plsc.load_gather(mid_v, [lo])
        return lo + jnp.where(mv > x, 1, 0)

    @pl.loop(0, NG)
    def _init(g):
        off = g * 16
        a = nearest(off, 0) * R
        for rp in range(R):
            v_ref[rp, pl.ds(off, 16)] = plsc.load_gather(cf_v, [a + rp])

    @pl.loop(0, DM, step=2)
    def _mid(d0):
        for sub in range(2):
            d = d0 + sub
            sem = sem0 if sub == 0 else sem1
            bufbase = sub * (R * M * R)
            pltpu.make_async_copy(
                cm_hbm.at[d], cm_v.at[pl.ds(bufbase, R * M * R)], sem).wait()
            nxt = d + 1

            @pl.when(nxt < DM)
            def _prefetch():
                nb = (sub ^ 1) * (R * M * R)
                nsem = sem1 if sub == 0 else sem0
                pltpu.async_copy(cm_hbm.at[nxt],
                                 cm_v.at[pl.ds(nb, R * M * R)], nsem)

            @pl.loop(0, NG)
            def _grp(g):
                off = g * 16
                rowa = nearest(off, d + 1) * R + bufbase
                acc = [None] * R
                for r in range(R):
                    ar = rowa + r * (M * R)
                    vr = v_ref[r, pl.ds(off, 16)]
                    for rp in range(R):
                        g_el = plsc.load_gather(cm_v, [ar + rp])
                        if r == 0:
                            acc[rp] = vr * g_el
                        else:
                            acc[rp] = acc[rp] + vr * g_el
                for rp in range(R):
                    v_ref[rp, pl.ds(off, 16)] = acc[rp]

    @pl.loop(0, NG)
    def _last(g):
        off = g * 16
        il = nearest(off, D - 1)
        acc = None
        for r in range(R):
            e = plsc.load_gather(cl_v, [il + r * M])
            t = v_ref[r, pl.ds(off, 16)] * e
            acc = t if acc is None else acc + t
        out_v[pl.ds(off, 16)] = acc + nn_v[pl.ds(off, 16)]

    pltpu.sync_copy(out_v, out_hbm.at[pl.ds(wid * P, P)])


def kernel(points, core_first, cores_mid, core_last, nodes, W1, b1, W2, b2, W3, b3):
    nn2 = pl.pallas_call(
        _tc_body,
        grid=(B // BT,),
        in_specs=[
            pl.BlockSpec((BT, D), lambda i: (i, 0)),
            pl.BlockSpec((H, D), lambda i: (0, 0)),
            pl.BlockSpec((1, H), lambda i: (0, 0)),
            pl.BlockSpec((H, H), lambda i: (0, 0)),
            pl.BlockSpec((1, H), lambda i: (0, 0)),
            pl.BlockSpec((1, H), lambda i: (0, 0)),
            pl.BlockSpec(memory_space=pltpu.SMEM),
        ],
        out_specs=pl.BlockSpec((BT, 1), lambda i: (i, 0)),
        out_shape=jax.ShapeDtypeStruct((B, 1), jnp.float32),
    )(points, W1, b1.reshape(1, H), W2, b2.reshape(1, H),
      W3, b3.reshape(1, 1))

    nodes1 = nodes[0]
    mids = (nodes1[:-1] + nodes1[1:]) * 0.5
    mid_pad = jnp.concatenate(
        [mids, jnp.full((1,), -1e30, jnp.float32)])
    edges = (jnp.arange(Q, dtype=jnp.float32) + 1.0) / Q
    lut = jnp.sum(mids[None, :] > edges[:, None], axis=1).astype(jnp.int32)

    ptsr = points.reshape(NW, P * D)
    nn2 = nn2.reshape(NW, P)
    cf_flat = core_first.reshape(M * R)
    cm2 = cores_mid.reshape(DM, R * M * R)
    cl_flat = core_last.reshape(R * M)

    mesh = plsc.VectorSubcoreMesh(core_axis_name="c", subcore_axis_name="s")
    out = pl.kernel(
        _sc_body,
        out_type=jax.ShapeDtypeStruct((B,), jnp.float32),
        mesh=mesh,
        compiler_params=pltpu.CompilerParams(needs_layout_passes=False),
        scratch_types=[
            pltpu.VMEM((P * D,), jnp.float32),
            pltpu.VMEM((P,), jnp.float32),
            pltpu.VMEM((Q,), jnp.int32),
            pltpu.VMEM((M,), jnp.float32),
            pltpu.VMEM((M * R,), jnp.float32),
            pltpu.VMEM((R * M,), jnp.float32),
            pltpu.VMEM((2 * R * M * R,), jnp.float32),
            pltpu.VMEM((R, P), jnp.float32),
            pltpu.VMEM((P,), jnp.float32),
            pltpu.SemaphoreType.DMA,
            pltpu.SemaphoreType.DMA,
        ],
    )(ptsr, nn2, lut, mid_pad, cf_flat, cm2, cl_flat)
    return out


# speedup vs baseline: 21.4137x; 2.0039x over previous
import jax
import jax.numpy as jnp
from jax import lax
from jax.experimental import pallas as pl
from jax.experimental.pallas import tpu as pltpu
from jax.experimental.pallas import tpu_sc as plsc

B = 16384
D = 26
M = 64
R = 16
H = 52
DM = D - 2
NC = 2
NS = 16
NW = NC * NS
P = B // NW
NG = P // 16
BT = 2048
Q = 4096


def _tc_body(x_ref, w1_ref, b1_ref, w2_ref, b2_ref, w3_ref, b3_ref, nn_ref):
    x = x_ref[...]
    cdims = (((1,), (1,)), ((), ()))
    h = jnp.maximum(
        lax.dot_general(x, w1_ref[...], cdims,
                        preferred_element_type=jnp.float32) + b1_ref[...], 0.0)
    h = jnp.maximum(
        lax.dot_general(h, w2_ref[...], cdims,
                        preferred_element_type=jnp.float32) + b2_ref[...], 0.0)
    nn_ref[...] = jnp.sum(h * w3_ref[...], axis=1, keepdims=True) + b3_ref[0, 0]


def _sc_body(pts_hbm, nn_hbm, lut_hbm, mid_hbm, cf_hbm, cm_hbm, cl_hbm, out_hbm,
             pts_v, nn_v, lut_v, mid_v, cf_v, cl_v, cm_v, v_ref, out_v,
             sem0, sem1):
    cid = lax.axis_index("c")
    sid = lax.axis_index("s")
    wid = sid * NC + cid
    pltpu.sync_copy(pts_hbm.at[wid], pts_v)
    pltpu.sync_copy(nn_hbm.at[wid], nn_v)
    pltpu.sync_copy(lut_hbm, lut_v)
    pltpu.sync_copy(mid_hbm, mid_v)
    pltpu.sync_copy(cf_hbm, cf_v)
    pltpu.sync_copy(cl_hbm, cl_v)
    pltpu.async_copy(cm_hbm.at[0], cm_v.at[pl.ds(0, R * M * R)], sem0)
    iota_d = lax.iota(jnp.int32, 16) * D

    def nearest(off, d):
        x = plsc.load_gather(pts_v, [iota_d + (off * D + d)])
        q = jnp.minimum((x * float(Q)).astype(jnp.int32), Q - 1)
        lo = plsc.load_gather(lut_v, [q])
        mv = plsc.load_gather(mid_v, [lo])
        return lo + jnp.where(mv > x, 1, 0)

    @pl.loop(0, NG)
    def _init(g):
        off = g * 16
        a = nearest(off, 0)
        for rp in range(R):
            v_ref[rp, pl.ds(off, 16)] = plsc.load_gather(cf_v, [a + rp * M])

    @pl.loop(0, DM, step=2)
    def _mid(d0):
        for sub in range(2):
            d = d0 + sub
            sem = sem0 if sub == 0 else sem1
            bufbase = sub * (R * M * R)
            pltpu.make_async_copy(
                cm_hbm.at[d], cm_v.at[pl.ds(bufbase, R * M * R)], sem).wait()
            nxt = d + 1

            @pl.when(nxt < DM)
            def _prefetch():
                nb = (sub ^ 1) * (R * M * R)
                nsem = sem1 if sub == 0 else sem0
                pltpu.async_copy(cm_hbm.at[nxt],
                                 cm_v.at[pl.ds(nb, R * M * R)], nsem)

            @pl.loop(0, NG)
            def _grp(g):
                off = g * 16
                rowa = nearest(off, d + 1) + bufbase
                acc = [None] * R
                for r in range(R):
                    ar = rowa + r * (R * M)
                    vr = v_ref[r, pl.ds(off, 16)]
                    for rp in range(R):
                        g_el = plsc.load_gather(cm_v, [ar + rp * M])
                        if r == 0:
                            acc[rp] = vr * g_el
                        else:
                            acc[rp] = acc[rp] + vr * g_el
                for rp in range(R):
                    v_ref[rp, pl.ds(off, 16)] = acc[rp]

    @pl.loop(0, NG)
    def _last(g):
        off = g * 16
        il = nearest(off, D - 1)
        acc = None
        for r in range(R):
            e = plsc.load_gather(cl_v, [il + r * M])
            t = v_ref[r, pl.ds(off, 16)] * e
            acc = t if acc is None else acc + t
        out_v[pl.ds(off, 16)] = acc + nn_v[pl.ds(off, 16)]

    pltpu.sync_copy(out_v, out_hbm.at[pl.ds(wid * P, P)])


def kernel(points, core_first, cores_mid, core_last, nodes, W1, b1, W2, b2, W3, b3):
    nn2 = pl.pallas_call(
        _tc_body,
        grid=(B // BT,),
        in_specs=[
            pl.BlockSpec((BT, D), lambda i: (i, 0)),
            pl.BlockSpec((H, D), lambda i: (0, 0)),
            pl.BlockSpec((1, H), lambda i: (0, 0)),
            pl.BlockSpec((H, H), lambda i: (0, 0)),
            pl.BlockSpec((1, H), lambda i: (0, 0)),
            pl.BlockSpec((1, H), lambda i: (0, 0)),
            pl.BlockSpec(memory_space=pltpu.SMEM),
        ],
        out_specs=pl.BlockSpec((BT, 1), lambda i: (i, 0)),
        out_shape=jax.ShapeDtypeStruct((B, 1), jnp.float32),
    )(points, W1, b1.reshape(1, H), W2, b2.reshape(1, H),
      W3, b3.reshape(1, 1))

    nodes1 = nodes[0]
    mids = (nodes1[:-1] + nodes1[1:]) * 0.5
    mid_pad = jnp.concatenate(
        [mids, jnp.full((1,), -1e30, jnp.float32)])
    edges = (jnp.arange(Q, dtype=jnp.float32) + 1.0) / Q
    lut = jnp.sum(mids[None, :] > edges[:, None], axis=1).astype(jnp.int32)

    ptsr = points.reshape(NW, P * D)
    nn2 = nn2.reshape(NW, P)
    cf_flat = core_first[0].T.reshape(R * M)
    cm2 = cores_mid.transpose(0, 1, 3, 2).reshape(DM, R * R * M)
    cl_flat = core_last.reshape(R * M)

    mesh = plsc.VectorSubcoreMesh(core_axis_name="c", subcore_axis_name="s")
    out = pl.kernel(
        _sc_body,
        out_type=jax.ShapeDtypeStruct((B,), jnp.float32),
        mesh=mesh,
        compiler_params=pltpu.CompilerParams(needs_layout_passes=False),
        scratch_types=[
            pltpu.VMEM((P * D,), jnp.float32),
            pltpu.VMEM((P,), jnp.float32),
            pltpu.VMEM((Q,), jnp.int32),
            pltpu.VMEM((M,), jnp.float32),
            pltpu.VMEM((M * R,), jnp.float32),
            pltpu.VMEM((R * M,), jnp.float32),
            pltpu.VMEM((2 * R * M * R,), jnp.float32),
            pltpu.VMEM((R, P), jnp.float32),
            pltpu.VMEM((P,), jnp.float32),
            pltpu.SemaphoreType.DMA,
            pltpu.SemaphoreType.DMA,
        ],
    )(ptsr, nn2, lut, mid_pad, cf_flat, cm2, cl_flat)
    return out
